# baseline (device time: 429459 ns/iter reference)
import jax
import jax.numpy as jnp
from jax import lax
from jax.experimental import pallas as pl
from jax.experimental.pallas import tpu as pltpu

N_DEV = 8
HEADS = 16
SEQ = 1024
HDIM = 128
SCALE = HDIM ** -0.5


def _body(q_ref, k_ref, v_ref, out_ref, cw_ref, ccw_ref,
          cw_send_sems, cw_recv_sems, ccw_send_sems, ccw_recv_sems,
          cw_credit, ccw_credit, l_ref, copy_sems):
    my = lax.axis_index("i")
    left = (my + N_DEV - 1) % N_DEV
    right = (my + 1) % N_DEV

    seeds = [
        pltpu.make_async_copy(k_ref, cw_ref.at[0, 0], copy_sems.at[0]),
        pltpu.make_async_copy(v_ref, cw_ref.at[0, 1], copy_sems.at[1]),
        pltpu.make_async_copy(k_ref, ccw_ref.at[0, 0], copy_sems.at[2]),
        pltpu.make_async_copy(v_ref, ccw_ref.at[0, 1], copy_sems.at[3]),
    ]
    for c in seeds:
        c.start()

    barrier_sem = pltpu.get_barrier_semaphore()
    for nbr in (left, right):
        pl.semaphore_signal(
            barrier_sem, inc=1,
            device_id=(nbr,), device_id_type=pl.DeviceIdType.MESH,
        )
    pl.semaphore_wait(barrier_sem, 2)
    for c in seeds:
        c.wait()

    l_ref[...] = jnp.zeros((HEADS, SEQ), jnp.float32)
    out_ref[...] = jnp.zeros((HEADS, SEQ, HDIM), jnp.float32)

    def accumulate(k_at, v_at):
        def head_body(h, carry):
            q = q_ref[h]
            k = k_at(h)
            v = v_at(h)
            s = lax.dot_general(
                q, k, (((1,), (1,)), ((), ())),
                preferred_element_type=jnp.float32,
            )
            p = jnp.exp(s.astype(jnp.bfloat16))
            l_ref[h] = l_ref[h] + jnp.sum(p, axis=1, dtype=jnp.float32)
            pv = lax.dot_general(
                p, v, (((1,), (0,)), ((), ())),
                preferred_element_type=jnp.float32,
            )
            out_ref[h] = out_ref[h] + pv
            return carry
        lax.fori_loop(0, HEADS, head_body, 0)

    def slot_chunk(ref, slot):
        return (lambda h: ref[slot, 0, h]), (lambda h: ref[slot, 1, h])

    for r in (1, 2, 3, 4):
        ss = (r - 1) % 2
        rs = r % 2
        if r >= 3:
            pl.semaphore_wait(cw_credit, 1)
            pl.semaphore_wait(ccw_credit, 1)
        if r < 4:
            cw_rdma = pltpu.make_async_remote_copy(
                src_ref=cw_ref.at[ss], dst_ref=cw_ref.at[rs],
                send_sem=cw_send_sems.at[ss], recv_sem=cw_recv_sems.at[rs],
                device_id=(right,), device_id_type=pl.DeviceIdType.MESH,
            )
            ccw_rdma = pltpu.make_async_remote_copy(
                src_ref=ccw_ref.at[ss], dst_ref=ccw_ref.at[rs],
                send_sem=ccw_send_sems.at[ss], recv_sem=ccw_recv_sems.at[rs],
                device_id=(left,), device_id_type=pl.DeviceIdType.MESH,
            )
        else:
            cw_rdma = pltpu.make_async_remote_copy(
                src_ref=cw_ref.at[ss, 0], dst_ref=cw_ref.at[rs, 0],
                send_sem=cw_send_sems.at[ss], recv_sem=cw_recv_sems.at[rs],
                device_id=(right,), device_id_type=pl.DeviceIdType.MESH,
            )
            ccw_rdma = pltpu.make_async_remote_copy(
                src_ref=ccw_ref.at[ss, 1], dst_ref=ccw_ref.at[rs, 1],
                send_sem=ccw_send_sems.at[ss], recv_sem=ccw_recv_sems.at[rs],
                device_id=(left,), device_id_type=pl.DeviceIdType.MESH,
            )
        cw_rdma.start()
        ccw_rdma.start()

        if r == 1:
            accumulate(*slot_chunk(cw_ref, 0))
        else:
            accumulate(*slot_chunk(cw_ref, ss))
            accumulate(*slot_chunk(ccw_ref, ss))
            if r < 4:
                pl.semaphore_signal(
                    cw_credit, inc=1,
                    device_id=(left,), device_id_type=pl.DeviceIdType.MESH,
                )
                pl.semaphore_signal(
                    ccw_credit, inc=1,
                    device_id=(right,), device_id_type=pl.DeviceIdType.MESH,
                )
        cw_rdma.wait()
        ccw_rdma.wait()

    accumulate(lambda h: cw_ref[0, 0, h], lambda h: ccw_ref[0, 1, h])

    def finalize(h, carry):
        out_ref[h] = out_ref[h] / l_ref[h][:, None]
        return carry
    lax.fori_loop(0, HEADS, finalize, 0)


def kernel(Q, K, V):
    qb = jnp.transpose(Q[0] * SCALE, (1, 0, 2)).astype(jnp.bfloat16)
    kb = jnp.transpose(K[0], (1, 0, 2)).astype(jnp.bfloat16)
    vb = jnp.transpose(V[0], (1, 0, 2)).astype(jnp.bfloat16)

    out = pl.pallas_call(
        _body,
        out_shape=jax.ShapeDtypeStruct((HEADS, SEQ, HDIM), jnp.float32),
        in_specs=[
            pl.BlockSpec(memory_space=pltpu.VMEM),
            pl.BlockSpec(memory_space=pl.ANY),
            pl.BlockSpec(memory_space=pl.ANY),
        ],
        out_specs=pl.BlockSpec(memory_space=pltpu.VMEM),
        scratch_shapes=[
            pltpu.VMEM((2, 2, HEADS, SEQ, HDIM), jnp.bfloat16),
            pltpu.VMEM((2, 2, HEADS, SEQ, HDIM), jnp.bfloat16),
            pltpu.SemaphoreType.DMA((2,)),
            pltpu.SemaphoreType.DMA((2,)),
            pltpu.SemaphoreType.DMA((2,)),
            pltpu.SemaphoreType.DMA((2,)),
            pltpu.SemaphoreType.REGULAR,
            pltpu.SemaphoreType.REGULAR,
            pltpu.VMEM((HEADS, SEQ), jnp.float32),
            pltpu.SemaphoreType.DMA((4,)),
        ],
        compiler_params=pltpu.CompilerParams(
            collective_id=0,
            vmem_limit_bytes=60 * 1024 * 1024,
        ),
    )(qb, kb, vb)
    return jnp.transpose(out, (1, 0, 2))[None]


# device time: 397567 ns/iter; 1.0802x vs baseline; 1.0802x over previous
import jax
import jax.numpy as jnp
from jax import lax
from jax.experimental import pallas as pl
from jax.experimental.pallas import tpu as pltpu

N_DEV = 8
HEADS = 16
SEQ = 1024
HDIM = 128
SCALE = HDIM ** -0.5


def _body(q_ref, k_ref, v_ref, out_ref, cw_ref, ccw_ref,
          cw_send_sems, cw_recv_sems, ccw_send_sems, ccw_recv_sems,
          cw_credit, ccw_credit, l_ref, copy_sems):
    my = lax.axis_index("i")
    left = (my + N_DEV - 1) % N_DEV
    right = (my + 1) % N_DEV

    seeds = [
        pltpu.make_async_copy(k_ref, cw_ref.at[0, 0], copy_sems.at[0]),
        pltpu.make_async_copy(v_ref, cw_ref.at[0, 1], copy_sems.at[1]),
        pltpu.make_async_copy(k_ref, ccw_ref.at[0, 0], copy_sems.at[2]),
        pltpu.make_async_copy(v_ref, ccw_ref.at[0, 1], copy_sems.at[3]),
    ]
    for c in seeds:
        c.start()

    barrier_sem = pltpu.get_barrier_semaphore()
    for nbr in (left, right):
        pl.semaphore_signal(
            barrier_sem, inc=1,
            device_id=(nbr,), device_id_type=pl.DeviceIdType.MESH,
        )
    pl.semaphore_wait(barrier_sem, 2)
    for c in seeds:
        c.wait()

    l_ref[...] = jnp.zeros((HEADS, SEQ), jnp.float32)
    out_ref[...] = jnp.zeros((HEADS, SEQ, HDIM), jnp.float32)

    HALF = SEQ // 2

    def accumulate(k_at, v_at):
        def head_body(h, carry):
            qh = q_ref[h]
            k = k_at(h)
            v = v_at(h)
            for lo in (0, HALF):
                q = qh[lo:lo + HALF]
                s = lax.dot_general(
                    q, k, (((1,), (1,)), ((), ())),
                    preferred_element_type=jnp.float32,
                )
                p = jnp.exp(s)
                l_ref[h, lo:lo + HALF] = (
                    l_ref[h, lo:lo + HALF] + jnp.sum(p, axis=1)
                )
                pv = lax.dot_general(
                    p.astype(jnp.bfloat16), v, (((1,), (0,)), ((), ())),
                    preferred_element_type=jnp.float32,
                )
                out_ref[h, lo:lo + HALF] = out_ref[h, lo:lo + HALF] + pv
            return carry
        lax.fori_loop(0, HEADS, head_body, 0)

    def slot_chunk(ref, slot):
        return (lambda h: ref[slot, 0, h]), (lambda h: ref[slot, 1, h])

    for r in (1, 2, 3, 4):
        ss = (r - 1) % 2
        rs = r % 2
        if r >= 3:
            pl.semaphore_wait(cw_credit, 1)
            pl.semaphore_wait(ccw_credit, 1)
        if r < 4:
            cw_rdma = pltpu.make_async_remote_copy(
                src_ref=cw_ref.at[ss], dst_ref=cw_ref.at[rs],
                send_sem=cw_send_sems.at[ss], recv_sem=cw_recv_sems.at[rs],
                device_id=(right,), device_id_type=pl.DeviceIdType.MESH,
            )
            ccw_rdma = pltpu.make_async_remote_copy(
                src_ref=ccw_ref.at[ss], dst_ref=ccw_ref.at[rs],
                send_sem=ccw_send_sems.at[ss], recv_sem=ccw_recv_sems.at[rs],
                device_id=(left,), device_id_type=pl.DeviceIdType.MESH,
            )
        else:
            cw_rdma = pltpu.make_async_remote_copy(
                src_ref=cw_ref.at[ss, 0], dst_ref=cw_ref.at[rs, 0],
                send_sem=cw_send_sems.at[ss], recv_sem=cw_recv_sems.at[rs],
                device_id=(right,), device_id_type=pl.DeviceIdType.MESH,
            )
            ccw_rdma = pltpu.make_async_remote_copy(
                src_ref=ccw_ref.at[ss, 1], dst_ref=ccw_ref.at[rs, 1],
                send_sem=ccw_send_sems.at[ss], recv_sem=ccw_recv_sems.at[rs],
                device_id=(left,), device_id_type=pl.DeviceIdType.MESH,
            )
        cw_rdma.start()
        ccw_rdma.start()

        if r == 1:
            accumulate(*slot_chunk(cw_ref, 0))
        else:
            accumulate(*slot_chunk(cw_ref, ss))
            accumulate(*slot_chunk(ccw_ref, ss))
            if r < 4:
                pl.semaphore_signal(
                    cw_credit, inc=1,
                    device_id=(left,), device_id_type=pl.DeviceIdType.MESH,
                )
                pl.semaphore_signal(
                    ccw_credit, inc=1,
                    device_id=(right,), device_id_type=pl.DeviceIdType.MESH,
                )
        cw_rdma.wait()
        ccw_rdma.wait()

    accumulate(lambda h: cw_ref[0, 0, h], lambda h: ccw_ref[0, 1, h])

    def finalize(h, carry):
        out_ref[h] = out_ref[h] / l_ref[h][:, None]
        return carry
    lax.fori_loop(0, HEADS, finalize, 0)


def kernel(Q, K, V):
    qb = jnp.transpose(Q[0] * SCALE, (1, 0, 2)).astype(jnp.bfloat16)
    kb = jnp.transpose(K[0], (1, 0, 2)).astype(jnp.bfloat16)
    vb = jnp.transpose(V[0], (1, 0, 2)).astype(jnp.bfloat16)

    out = pl.pallas_call(
        _body,
        out_shape=jax.ShapeDtypeStruct((HEADS, SEQ, HDIM), jnp.float32),
        in_specs=[
            pl.BlockSpec(memory_space=pltpu.VMEM),
            pl.BlockSpec(memory_space=pl.ANY),
            pl.BlockSpec(memory_space=pl.ANY),
        ],
        out_specs=pl.BlockSpec(memory_space=pltpu.VMEM),
        scratch_shapes=[
            pltpu.VMEM((2, 2, HEADS, SEQ, HDIM), jnp.bfloat16),
            pltpu.VMEM((2, 2, HEADS, SEQ, HDIM), jnp.bfloat16),
            pltpu.SemaphoreType.DMA((2,)),
            pltpu.SemaphoreType.DMA((2,)),
            pltpu.SemaphoreType.DMA((2,)),
            pltpu.SemaphoreType.DMA((2,)),
            pltpu.SemaphoreType.REGULAR,
            pltpu.SemaphoreType.REGULAR,
            pltpu.VMEM((HEADS, SEQ), jnp.float32),
            pltpu.SemaphoreType.DMA((4,)),
        ],
        compiler_params=pltpu.CompilerParams(
            collective_id=0,
            vmem_limit_bytes=60 * 1024 * 1024,
        ),
    )(qb, kb, vb)
    return jnp.transpose(out, (1, 0, 2))[None]


# device time: 396302 ns/iter; 1.0837x vs baseline; 1.0032x over previous
import jax
import jax.numpy as jnp
from jax import lax
from jax.experimental import pallas as pl
from jax.experimental.pallas import tpu as pltpu

N_DEV = 8
HEADS = 16
SEQ = 1024
HDIM = 128
SCALE = HDIM ** -0.5


def _body(q_ref, k_ref, v_ref, out_ref, cw_ref, ccw_ref,
          cw_send_sems, cw_recv_sems, ccw_send_sems, ccw_recv_sems,
          cw_credit, ccw_credit, l_ref, copy_sems):
    my = lax.axis_index("i")
    left = (my + N_DEV - 1) % N_DEV
    right = (my + 1) % N_DEV

    seeds = [
        pltpu.make_async_copy(k_ref, cw_ref.at[0, 0], copy_sems.at[0]),
        pltpu.make_async_copy(v_ref, cw_ref.at[0, 1], copy_sems.at[1]),
    ]
    for c in seeds:
        c.start()

    barrier_sem = pltpu.get_barrier_semaphore()
    for nbr in (left, right):
        pl.semaphore_signal(
            barrier_sem, inc=1,
            device_id=(nbr,), device_id_type=pl.DeviceIdType.MESH,
        )
    pl.semaphore_wait(barrier_sem, 2)
    for c in seeds:
        c.wait()

    l_ref[...] = jnp.zeros((HEADS, SEQ), jnp.float32)
    out_ref[...] = jnp.zeros((HEADS, SEQ, HDIM), jnp.float32)

    HALF = SEQ // 2

    def accumulate(k_at, v_at, last=False):
        def head_body(h, carry):
            qh = q_ref[h]
            k = k_at(h)
            v = v_at(h)
            for lo in (0, HALF):
                q = qh[lo:lo + HALF]
                s = lax.dot_general(
                    q, k, (((1,), (1,)), ((), ())),
                    preferred_element_type=jnp.float32,
                )
                p = jnp.exp(s)
                l_new = l_ref[h, lo:lo + HALF] + jnp.sum(p, axis=1)
                pv = lax.dot_general(
                    p.astype(jnp.bfloat16), v, (((1,), (0,)), ((), ())),
                    preferred_element_type=jnp.float32,
                )
                acc = out_ref[h, lo:lo + HALF] + pv
                if last:
                    acc = acc * (1.0 / l_new)[:, None]
                else:
                    l_ref[h, lo:lo + HALF] = l_new
                out_ref[h, lo:lo + HALF] = acc
            return carry
        lax.fori_loop(0, HEADS, head_body, 0)

    def slot_chunk(ref, slot):
        return (lambda h: ref[slot, 0, h]), (lambda h: ref[slot, 1, h])

    for r in (1, 2, 3, 4):
        ss = (r - 1) % 2
        rs = r % 2
        if r >= 3:
            pl.semaphore_wait(cw_credit, 1)
            pl.semaphore_wait(ccw_credit, 1)
        if r < 4:
            cw_rdma = pltpu.make_async_remote_copy(
                src_ref=cw_ref.at[ss], dst_ref=cw_ref.at[rs],
                send_sem=cw_send_sems.at[ss], recv_sem=cw_recv_sems.at[rs],
                device_id=(right,), device_id_type=pl.DeviceIdType.MESH,
            )
            ccw_src = cw_ref.at[0] if r == 1 else ccw_ref.at[ss]
            ccw_rdma = pltpu.make_async_remote_copy(
                src_ref=ccw_src, dst_ref=ccw_ref.at[rs],
                send_sem=ccw_send_sems.at[ss], recv_sem=ccw_recv_sems.at[rs],
                device_id=(left,), device_id_type=pl.DeviceIdType.MESH,
            )
        else:
            cw_rdma = pltpu.make_async_remote_copy(
                src_ref=cw_ref.at[ss, 0], dst_ref=cw_ref.at[rs, 0],
                send_sem=cw_send_sems.at[ss], recv_sem=cw_recv_sems.at[rs],
                device_id=(right,), device_id_type=pl.DeviceIdType.MESH,
            )
            ccw_rdma = pltpu.make_async_remote_copy(
                src_ref=ccw_ref.at[ss, 1], dst_ref=ccw_ref.at[rs, 1],
                send_sem=ccw_send_sems.at[ss], recv_sem=ccw_recv_sems.at[rs],
                device_id=(left,), device_id_type=pl.DeviceIdType.MESH,
            )
        cw_rdma.start()
        ccw_rdma.start()

        if r == 1:
            accumulate(*slot_chunk(cw_ref, 0))
        else:
            accumulate(*slot_chunk(cw_ref, ss))
            accumulate(*slot_chunk(ccw_ref, ss))
            if r < 4:
                pl.semaphore_signal(
                    cw_credit, inc=1,
                    device_id=(left,), device_id_type=pl.DeviceIdType.MESH,
                )
                pl.semaphore_signal(
                    ccw_credit, inc=1,
                    device_id=(right,), device_id_type=pl.DeviceIdType.MESH,
                )
        cw_rdma.wait()
        ccw_rdma.wait()

    accumulate(lambda h: cw_ref[0, 0, h], lambda h: ccw_ref[0, 1, h],
               last=True)


def kernel(Q, K, V):
    qb = jnp.transpose(Q[0] * SCALE, (1, 0, 2)).astype(jnp.bfloat16)
    kb = jnp.transpose(K[0], (1, 0, 2)).astype(jnp.bfloat16)
    vb = jnp.transpose(V[0], (1, 0, 2)).astype(jnp.bfloat16)

    out = pl.pallas_call(
        _body,
        out_shape=jax.ShapeDtypeStruct((HEADS, SEQ, HDIM), jnp.float32),
        in_specs=[
            pl.BlockSpec(memory_space=pltpu.VMEM),
            pl.BlockSpec(memory_space=pl.ANY),
            pl.BlockSpec(memory_space=pl.ANY),
        ],
        out_specs=pl.BlockSpec(memory_space=pltpu.VMEM),
        scratch_shapes=[
            pltpu.VMEM((2, 2, HEADS, SEQ, HDIM), jnp.bfloat16),
            pltpu.VMEM((2, 2, HEADS, SEQ, HDIM), jnp.bfloat16),
            pltpu.SemaphoreType.DMA((2,)),
            pltpu.SemaphoreType.DMA((2,)),
            pltpu.SemaphoreType.DMA((2,)),
            pltpu.SemaphoreType.DMA((2,)),
            pltpu.SemaphoreType.REGULAR,
            pltpu.SemaphoreType.REGULAR,
            pltpu.VMEM((HEADS, SEQ), jnp.float32),
            pltpu.SemaphoreType.DMA((2,)),
        ],
        compiler_params=pltpu.CompilerParams(
            collective_id=0,
            vmem_limit_bytes=60 * 1024 * 1024,
        ),
    )(qb, kb, vb)
    return jnp.transpose(out, (1, 0, 2))[None]
